# Initial kernel scaffold; baseline (speedup 1.0000x reference)
import jax
import jax.numpy as jnp
from jax.experimental import pallas as pl


def kernel(xyz, feat, params):
    raise NotImplementedError("write your pallas kernel here")


# speedup vs baseline: 2.4145x; 2.4145x over previous
import functools

import jax
import jax.numpy as jnp
from jax import lax
from jax.experimental import pallas as pl

_CFGS = [
    dict(npoint=1024, nsample=38, in_ch=3, mlp=[32, 32, 64], bw=0.05),
    dict(npoint=256, nsample=38, in_ch=67, mlp=[64, 64, 128], bw=0.10),
    dict(npoint=64, nsample=38, in_ch=131, mlp=[128, 128, 256], bw=0.20),
    dict(npoint=36, nsample=38, in_ch=259, mlp=[256, 256, 512], bw=0.40),
]

_INTERPRET = False


def _density_body(xc_ref, rows_ref, out_ref, *, bw):
    a = rows_ref[...]
    xc = xc_ref[...]
    ab = lax.dot_general(a, xc, (((1,), (0,)), ((), ())),
                         preferred_element_type=jnp.float32)
    na = jnp.sum(a * a, axis=1, keepdims=True)
    nb = jnp.sum(xc * xc, axis=0, keepdims=True)
    d = -2.0 * ab + na + nb
    g = jnp.exp(-d / (2.0 * bw * bw)) / (2.5 * bw)
    out_ref[...] = jnp.mean(g, axis=-1)[None, :]


def _density(xyz_c, bw):
    B, _, N = xyz_c.shape
    xyz_t = jnp.transpose(xyz_c, (0, 2, 1))
    blk = min(N, 512)
    out = pl.pallas_call(
        functools.partial(_density_body, bw=bw),
        grid=(B, N // blk),
        in_specs=[pl.BlockSpec((None, 3, N), lambda b, i: (b, 0, 0)),
                  pl.BlockSpec((None, blk, 3), lambda b, i: (b, i, 0))],
        out_specs=pl.BlockSpec((None, 1, blk), lambda b, i: (b, 0, i)),
        out_shape=jax.ShapeDtypeStruct((B, 1, N), jnp.float32),
        interpret=_INTERPRET,
    )(xyz_c, xyz_t)
    return out.reshape(B, N)


def _fps_body(xc_ref, out_ref, *, npoint):
    b, _, n = xc_ref.shape
    xs = [xc_ref[:, c, :] for c in range(3)]
    iota = lax.broadcasted_iota(jnp.int32, (b, n), 1)

    iota_p = lax.broadcasted_iota(jnp.int32, (b, npoint), 1)

    def body(i, state):
        cent, dist, far = state
        cent = jnp.where(iota_p == i, jnp.broadcast_to(far, cent.shape), cent)
        oh = (iota == far).astype(jnp.float32)
        d = None
        for c in range(3):
            cc = jnp.sum(xs[c] * oh, axis=-1, keepdims=True)
            t = xs[c] - cc
            t = t * t
            d = t if d is None else d + t
        dist = jnp.minimum(dist, d)
        far = jnp.argmax(dist, axis=-1, keepdims=True).astype(jnp.int32)
        return cent, dist, far

    cent0 = out_ref[...]
    dist0 = xs[0] * 0.0 + 1e10
    far0 = cent0[:, :1] * 0
    cent, _, _ = lax.fori_loop(0, npoint, body, (cent0, dist0, far0))
    out_ref[...] = cent


def _fps(xyz_c, npoint):
    B = xyz_c.shape[0]
    return pl.pallas_call(
        functools.partial(_fps_body, npoint=npoint),
        out_shape=jax.ShapeDtypeStruct((B, npoint), jnp.int32),
        interpret=_INTERPRET,
    )(xyz_c)


def _knn_body(xc_ref, new_ref, out_ref, *, k):
    a = new_ref[...]
    xc = xc_ref[...]
    sblk = a.shape[0]
    n = xc.shape[1]
    ab = lax.dot_general(a, xc, (((1,), (0,)), ((), ())),
                         preferred_element_type=jnp.float32)
    na = jnp.sum(a * a, axis=1, keepdims=True)
    nb = jnp.sum(xc * xc, axis=0, keepdims=True)
    d = -2.0 * ab + na + nb
    iot = lax.broadcasted_iota(jnp.int32, (sblk, n), 1)

    iota_k = lax.broadcasted_iota(jnp.int32, (sblk, k), 1)

    def body(kk, state):
        d, out = state
        am = jnp.argmin(d, axis=-1, keepdims=True).astype(jnp.int32)
        out = jnp.where(iota_k == kk, jnp.broadcast_to(am, out.shape), out)
        d = jnp.where(iot == am, jnp.inf, d)
        return d, out

    out0 = out_ref[...]
    _, out = lax.fori_loop(0, k, body, (d, out0))
    out_ref[...] = out


def _knn(new_xyz, xyz_c, k):
    B, S, _ = new_xyz.shape
    N = xyz_c.shape[2]
    sblk = min(S, 256)
    return pl.pallas_call(
        functools.partial(_knn_body, k=k),
        grid=(B, S // sblk),
        in_specs=[pl.BlockSpec((None, 3, N), lambda b, i: (b, 0, 0)),
                  pl.BlockSpec((None, sblk, 3), lambda b, i: (b, i, 0))],
        out_specs=pl.BlockSpec((None, sblk, k), lambda b, i: (b, i, 0)),
        out_shape=jax.ShapeDtypeStruct((B, S, k), jnp.int32),
        interpret=_INTERPRET,
    )(xyz_c, new_xyz)


def _index_points(points, idx):
    return jax.vmap(lambda p, i: p[i])(points, idx)


def _conv_bn(x, p, act):
    y = jnp.einsum('bcks,oc->boks', x, p['W']) + p['b'][None, :, None, None]
    mean = jnp.mean(y, axis=(0, 2, 3), keepdims=True)
    var = jnp.var(y, axis=(0, 2, 3), keepdims=True)
    y = (y - mean) / jnp.sqrt(var + 1e-5)
    y = y * p['gamma'][None, :, None, None] + p['beta'][None, :, None, None]
    return jax.nn.relu(y) if act == 'relu' else jax.nn.sigmoid(y)


def _sa(xyz_c, points, p, cfg):
    B = xyz_c.shape[0]
    xyz_t = jnp.transpose(xyz_c, (0, 2, 1))
    pts_t = jnp.transpose(points, (0, 2, 1))
    density = _density(xyz_c, cfg['bw'])
    inv_density = (1.0 / density)[:, :, None]
    S = cfg['npoint']
    K = cfg['nsample']
    fps_idx = _fps(xyz_c, S)
    new_xyz = _index_points(xyz_t, fps_idx)
    idx = _knn(new_xyz, xyz_c, K)
    grouped_xyz = _index_points(xyz_t, idx)
    grouped_xyz_norm = grouped_xyz - new_xyz[:, :, None, :]
    grouped_points = _index_points(pts_t, idx)
    new_points = jnp.concatenate([grouped_xyz_norm, grouped_points], axis=-1)
    grouped_density = _index_points(inv_density, idx)
    x = jnp.transpose(new_points, (0, 3, 2, 1))
    for cp in p['mlp']:
        x = _conv_bn(x, cp, 'relu')
    inv_max = jnp.max(grouped_density, axis=2, keepdims=True)
    dscale = grouped_density / inv_max
    d = jnp.transpose(dscale, (0, 3, 2, 1))
    nd = len(p['densitynet'])
    for j, cp in enumerate(p['densitynet']):
        d = _conv_bn(d, cp, 'sigmoid' if j == nd - 1 else 'relu')
    x = x * d
    g = jnp.transpose(grouped_xyz_norm, (0, 3, 2, 1))
    for cp in p['weightnet']:
        g = _conv_bn(g, cp, 'relu')
    xm = jnp.transpose(x, (0, 3, 1, 2))
    gm = jnp.transpose(g, (0, 3, 2, 1))
    out = jnp.matmul(xm, gm).reshape(B, S, -1)
    out = out @ p['linear']['W'].T + p['linear']['b']
    out = jnp.transpose(out, (0, 2, 1))
    mean = jnp.mean(out, axis=(0, 2), keepdims=True)
    var = jnp.var(out, axis=(0, 2), keepdims=True)
    out = (out - mean) / jnp.sqrt(var + 1e-5)
    out = out * p['bn_linear']['gamma'][None, :, None] + p['bn_linear']['beta'][None, :, None]
    out = jax.nn.relu(out)
    return jnp.transpose(new_xyz, (0, 2, 1)), out


def kernel(xyz, feat, params):
    l1x, l1p = _sa(xyz, feat, params['sa1'], _CFGS[0])
    l2x, l2p = _sa(l1x, l1p, params['sa2'], _CFGS[1])
    l3x, l3p = _sa(l2x, l2p, params['sa3'], _CFGS[2])
    l4x, l4p = _sa(l3x, l3p, params['sa4'], _CFGS[3])
    return (l1x, l1p, l2x, l2p, l3x, l3p, l4x, l4p)
